# baseline (speedup 1.0000x reference)
import functools

import jax
import jax.numpy as jnp
from jax import lax
from jax.experimental import pallas as pl
from jax.experimental.pallas import tpu as pltpu
from jax.experimental.pallas import tpu_sc as plsc

_B = 16384
_F = 26
_V = 100000
_D = 32
_NC = 2
_NS = 16
_NW = _NC * _NS
_Q = 4096
_NQ = _B // _Q
_PARTS = ((7, 0), (7, 7), (6, 14), (6, 20))

_mesh = plsc.VectorSubcoreMesh(
    core_axis_name="c", subcore_axis_name="s",
    num_cores=_NC, num_subcores=_NS)


@functools.cache
def _make_plane_kernel(fp):
    @functools.partial(
        pl.kernel,
        out_type=jax.ShapeDtypeStruct((fp * _D * _B,), jnp.float32),
        mesh=_mesh,
        scratch_types=[
            pltpu.VMEM((_V,), jnp.float32),
            pltpu.VMEM((_B,), jnp.int32),
            pltpu.VMEM((2, _Q), jnp.float32),
            pltpu.SemaphoreType.DMA,
        ],
        compiler_params=pltpu.CompilerParams(
            use_tc_tiling_on_sc=False, needs_layout_passes=False),
    )
    def plane_kernel(tab_hbm, idx_hbm, out_hbm, vec_v, idx_v, out_v, wsem):
        wid = lax.axis_index("s") * _NC + lax.axis_index("c")
        item0 = wid * fp

        def out_cp(item, q, buf):
            return pltpu.make_async_copy(
                out_v.at[buf], out_hbm.at[pl.ds(item * _B + q * _Q, _Q)],
                wsem)

        def body(i, prev_f):
            item = item0 + i
            f = item // _D
            d = item % _D

            @pl.when(f != prev_f)
            def _():
                pltpu.sync_copy(idx_hbm.at[pl.ds(f * _B, _B)], idx_v)

            pltpu.sync_copy(tab_hbm.at[pl.ds(item * _V, _V)], vec_v)

            for q in range(_NQ):
                buf = q % 2

                def gather_body(g, carry):
                    b0 = q * _Q + g * 64
                    for u in range(4):
                        idx16 = idx_v[pl.ds(b0 + u * 16, 16)]
                        out_v[buf, pl.ds(g * 64 + u * 16, 16)] = (
                            plsc.load_gather(vec_v, [idx16]))
                    return carry

                lax.fori_loop(0, _Q // 64, gather_body, None)
                if q >= 2:
                    out_cp(item, q - 2, buf).wait()
                out_cp(item, q, buf).start()

            out_cp(item, _NQ - 2, 0).wait()
            out_cp(item, _NQ - 1, 1).wait()
            return f

        lax.fori_loop(0, fp, body, jnp.int32(-1))

    return plane_kernel


def kernel(indices, tables):
    outs = []
    for fp, foff in _PARTS:
        tab_part = jnp.transpose(
            tables[foff:foff + fp], (0, 2, 1)).reshape(fp * _D * _V)
        idx_part = jnp.transpose(
            indices[:, foff:foff + fp], (1, 0)).reshape(fp * _B)
        out_part = _make_plane_kernel(fp)(tab_part, idx_part)
        outs.append(out_part.reshape(fp * _D, _B))
    cat = jnp.concatenate(outs, axis=0)
    return jnp.transpose(cat, (1, 0)).reshape(_B, _F * _D)


# speedup vs baseline: 1.1290x; 1.1290x over previous
import functools

import jax
import jax.numpy as jnp
from jax import lax
from jax.experimental import pallas as pl
from jax.experimental.pallas import tpu as pltpu
from jax.experimental.pallas import tpu_sc as plsc

_B = 16384
_F = 26
_V = 100000
_D = 32
_NC = 2
_NS = 16
_NW = _NC * _NS
_Q = 4096
_NQ = _B // _Q
_PARTS = ((26, 0),)

_mesh = plsc.VectorSubcoreMesh(
    core_axis_name="c", subcore_axis_name="s",
    num_cores=_NC, num_subcores=_NS)


@functools.cache
def _make_plane_kernel(fp):
    @functools.partial(
        pl.kernel,
        out_type=jax.ShapeDtypeStruct((fp * _D * _B,), jnp.float32),
        mesh=_mesh,
        scratch_types=[
            pltpu.VMEM((_V,), jnp.float32),
            pltpu.VMEM((_B,), jnp.int32),
            pltpu.VMEM((2, _Q), jnp.float32),
            pltpu.SemaphoreType.DMA,
        ],
        compiler_params=pltpu.CompilerParams(
            use_tc_tiling_on_sc=False, needs_layout_passes=False),
    )
    def plane_kernel(tab_hbm, idx_hbm, out_hbm, vec_v, idx_v, out_v, wsem):
        wid = lax.axis_index("s") * _NC + lax.axis_index("c")
        item0 = wid * fp

        def out_cp(item, q, buf):
            return pltpu.make_async_copy(
                out_v.at[buf], out_hbm.at[pl.ds(item * _B + q * _Q, _Q)],
                wsem)

        def body(i, prev_f):
            item = item0 + i
            f = item // _D
            d = item % _D

            @pl.when(f != prev_f)
            def _():
                pltpu.sync_copy(idx_hbm.at[pl.ds(f * _B, _B)], idx_v)

            pltpu.sync_copy(tab_hbm.at[pl.ds(item * _V, _V)], vec_v)

            for q in range(_NQ):
                buf = q % 2

                def gather_body(g, carry):
                    b0 = q * _Q + g * 256
                    for u in range(16):
                        idx16 = idx_v[pl.ds(b0 + u * 16, 16)]
                        out_v[buf, pl.ds(g * 256 + u * 16, 16)] = (
                            plsc.load_gather(vec_v, [idx16]))
                    return carry

                lax.fori_loop(0, _Q // 256, gather_body, None)
                if q >= 2:
                    out_cp(item, q - 2, buf).wait()
                out_cp(item, q, buf).start()

            out_cp(item, _NQ - 2, 0).wait()
            out_cp(item, _NQ - 1, 1).wait()
            return f

        lax.fori_loop(0, fp, body, jnp.int32(-1))

    return plane_kernel


def kernel(indices, tables):
    tab_t = jnp.transpose(tables, (0, 2, 1))
    idx_t = jnp.transpose(indices, (1, 0))
    outs = []
    for fp, foff in _PARTS:
        tab_part = tab_t[foff:foff + fp].reshape(fp * _D * _V)
        idx_part = idx_t[foff:foff + fp].reshape(fp * _B)
        out_part = _make_plane_kernel(fp)(tab_part, idx_part)
        outs.append(out_part.reshape(fp * _D, _B))
    cat = jnp.concatenate(outs, axis=0)
    return jnp.transpose(cat, (1, 0)).reshape(_B, _F * _D)
